# Initial kernel scaffold; baseline (speedup 1.0000x reference)
import functools

import jax
import jax.numpy as jnp
from jax import lax
from jax.experimental import pallas as pl
from jax.experimental.pallas import tpu as pltpu
from jax.experimental.pallas import tpu_sc as plsc

K = 16
NB = 8
BQ = 128
WT = 512
BD = 200
B1 = 200
TW = 272
BIG = 1e10


def _ln(t, g, b):
    m = jnp.mean(t, axis=-1, keepdims=True)
    d = t - m
    v = jnp.mean(d * d, axis=-1, keepdims=True)
    return d * lax.rsqrt(v + 1e-5) * g + b


def _elu(t):
    return jnp.where(t > 0, t, jnp.expm1(t))


def _proj_body(x_ref, posp_ref, wl_ref, bl_ref, ws_ref, tab_ref):
    x = x_ref[...]
    xl = jnp.dot(x, wl_ref[...], preferred_element_type=jnp.float32) + bl_ref[...]
    asrc = jnp.dot(x, ws_ref[...], preferred_element_type=jnp.float32)
    tab_ref[:, 0:128] = xl
    tab_ref[:, 128:256] = asrc
    tab_ref[:, 256:272] = posp_ref[...]


def _knn_body(lo_ref, nt_ref, posq_ref, qb_ref, posT_ref, kb_ref,
              out_ref, d2_ref):
    i = pl.program_id(0)
    lo = lo_ref[i]
    nt = nt_ref[i]
    posq = posq_ref[...]
    sqq = jnp.sum(posq * posq, axis=1, keepdims=True)
    qb = qb_ref[...]
    qrow = lax.broadcasted_iota(jnp.int32, (BQ, 1), 0) + i * BQ

    def fill(t, _):
        pT = posT_ref[t]
        qk = jnp.dot(posq, pT, preferred_element_type=jnp.float32)
        sqk = jnp.sum(pT * pT, axis=0, keepdims=True)
        d2 = sqq + sqk - 2.0 * qk
        gcol = lax.broadcasted_iota(jnp.int32, (BQ, WT), 1) + t * WT
        bad = (kb_ref[t] != qb) | (gcol == qrow)
        d2_ref[t] = jnp.where(bad, jnp.float32(BIG), d2)
        return 0

    lax.fori_loop(lo, lo + nt, fill, 0)

    imax = jnp.int32(2**30)
    prevv = jnp.full((BQ, 1), -jnp.inf, jnp.float32)
    previ = jnp.full((BQ, 1), -1, jnp.int32)
    for k in range(K):
        def scan(t, carry, prevv=prevv, previ=previ):
            bv, bi = carry
            v = d2_ref[t]
            gcol = lax.broadcasted_iota(jnp.int32, (BQ, WT), 1) + t * WT
            elig = (v > prevv) | ((v == prevv) & (gcol > previ))
            vv = jnp.where(elig, v, jnp.inf)
            tmin = jnp.min(vv, axis=1, keepdims=True)
            ii = jnp.min(jnp.where(elig & (v == tmin), gcol, imax),
                         axis=1, keepdims=True)
            better = (tmin < bv) | ((tmin == bv) & (ii < bi))
            return (jnp.where(better, tmin, bv), jnp.where(better, ii, bi))

        bv, bi = lax.fori_loop(
            lo, lo + nt, scan,
            (jnp.full((BQ, 1), jnp.inf, jnp.float32),
             jnp.full((BQ, 1), imax, jnp.int32)))
        out_ref[:, k:k + 1] = bi
        prevv, previ = bv, bi


def _gather_rows(table, idx3, n_edges):
    rows = n_edges // 128
    trips = (rows + 31) // 32
    mesh = plsc.VectorSubcoreMesh(core_axis_name="c", subcore_axis_name="s")

    @functools.partial(
        pl.kernel, mesh=mesh,
        out_type=jax.ShapeDtypeStruct((n_edges, TW), jnp.float32),
        scratch_types=[
            pltpu.VMEM((trips, 128), jnp.int32),
            pltpu.VMEM((128, TW), jnp.float32),
            pltpu.SemaphoreType.DMA,
        ])
    def gk(tab_hbm, idx_hbm, out_hbm, idx_v, buf, sem):
        w = lax.axis_index("s") * 2 + lax.axis_index("c")
        pltpu.sync_copy(idx_hbm.at[pl.ds(w * trips, trips)], idx_v)
        nrow = jnp.minimum(trips, rows - w * trips)

        def body(j, _):
            row = w * trips + j
            pltpu.async_copy(tab_hbm.at[idx_v.at[j]], buf, sem).wait()
            pltpu.sync_copy(buf, out_hbm.at[pl.ds(row * 128, 128)])
            return 0

        lax.fori_loop(0, nrow, body, 0)

    return gk(table, idx3)


def _edge_body(x_ref, posp_ref, g_ref, wd_ref,
               pw1_ref, pb1_ref, pg1_ref, pbe1_ref,
               pw2_ref, pb2_ref, pg2_ref, pbe2_ref,
               aw1_ref, ab1_ref, ag1_ref, abe1_ref,
               aw2_ref, ab2_ref, ag2_ref, abe2_ref,
               out_ref, a_scr, m_scr):
    x = x_ref[...]
    adst = jnp.dot(x, wd_ref[...], preferred_element_type=jnp.float32)
    posp = posp_ref[...]

    def p1(k, rmax):
        gk = g_ref[k]
        gxl = gk[:, 0:128]
        gas = gk[:, 128:256]
        gpp = gk[:, 256:272]
        rel = posp - gpp
        h = _elu(_ln(jnp.dot(rel, pw1_ref[...],
                             preferred_element_type=jnp.float32) + pb1_ref[...],
                     pg1_ref[...], pbe1_ref[...]))
        delta = _elu(_ln(jnp.dot(h, pw2_ref[...],
                                 preferred_element_type=jnp.float32) + pb2_ref[...],
                         pg2_ref[...], pbe2_ref[...]))
        ai = adst - gas + delta
        h2 = _elu(_ln(jnp.dot(ai, aw1_ref[...],
                              preferred_element_type=jnp.float32) + ab1_ref[...],
                      ag1_ref[...], abe1_ref[...]))
        al = _elu(_ln(jnp.dot(h2, aw2_ref[...],
                              preferred_element_type=jnp.float32) + ab2_ref[...],
                      ag2_ref[...], abe2_ref[...]))
        a_scr[k] = al
        m_scr[k] = gxl + delta
        return jnp.maximum(rmax, al)

    rmax = lax.fori_loop(0, K, p1, jnp.full((BD, 128), -jnp.inf, jnp.float32))

    def p2(k, carry):
        s, acc = carry
        e = jnp.exp(a_scr[k] - rmax)
        return s + e, acc + e * m_scr[k]

    s, acc = lax.fori_loop(
        0, K, p2,
        (jnp.zeros((BD, 128), jnp.float32), jnp.zeros((BD, 128), jnp.float32)))
    out_ref[...] = acc / s


def kernel(x, pos, batch, params):
    n = x.shape[0]
    npad = ((n + WT - 1) // WT) * WT
    nq = npad // BQ
    nt = npad // WT

    batch_i = batch.astype(jnp.int32)
    posp = jnp.pad(pos, ((0, 0), (0, 13)))
    pos8 = jnp.pad(pos, ((0, npad - n), (0, 5)))
    posT3 = pos8.T.reshape(8, nt, WT).transpose(1, 0, 2)
    kb3 = jnp.pad(batch_i, (0, npad - n),
                  constant_values=-1).reshape(nt, 1, WT)
    qb2 = jnp.pad(batch_i, (0, npad - n),
                  constant_values=-2).reshape(npad, 1)

    seg = jnp.searchsorted(batch_i, jnp.arange(NB + 1, dtype=jnp.int32),
                           side='left').astype(jnp.int32)
    starts = jnp.arange(nq, dtype=jnp.int32) * BQ
    first = jnp.minimum(starts, n - 1)
    last = jnp.minimum(starts + BQ - 1, n - 1)
    clo = seg[batch_i[first]]
    chi = seg[batch_i[last] + 1]
    lo_t = clo // WT
    n_t = jnp.where(starts >= n, 0, (chi + WT - 1) // WT - lo_t)

    table = pl.pallas_call(
        _proj_body,
        grid=(n // B1,),
        in_specs=[
            pl.BlockSpec((B1, 128), lambda i: (i, 0)),
            pl.BlockSpec((B1, 16), lambda i: (i, 0)),
            pl.BlockSpec((128, 128), lambda i: (0, 0)),
            pl.BlockSpec((1, 128), lambda i: (0, 0)),
            pl.BlockSpec((128, 128), lambda i: (0, 0)),
        ],
        out_specs=pl.BlockSpec((B1, TW), lambda i: (i, 0)),
        out_shape=jax.ShapeDtypeStruct((n, TW), jnp.float32),
    )(x, posp, params['W_lin'], params['b_lin'].reshape(1, 128),
      params['W_src'])

    idxmat = pl.pallas_call(
        _knn_body,
        grid=(nq,),
        in_specs=[
            pl.BlockSpec(memory_space=pltpu.SMEM),
            pl.BlockSpec(memory_space=pltpu.SMEM),
            pl.BlockSpec((BQ, 8), lambda i: (i, 0)),
            pl.BlockSpec((BQ, 1), lambda i: (i, 0)),
            pl.BlockSpec((nt, 8, WT), lambda i: (0, 0, 0)),
            pl.BlockSpec((nt, 1, WT), lambda i: (0, 0, 0)),
        ],
        out_specs=pl.BlockSpec((BQ, K), lambda i: (i, 0)),
        out_shape=jax.ShapeDtypeStruct((npad, K), jnp.int32),
        scratch_shapes=[pltpu.VMEM((nt, BQ, WT), jnp.float32)],
    )(lo_t, n_t, pos8, qb2, posT3, kb3)

    n_edges = K * n
    rows = n_edges // 128
    trips = (rows + 31) // 32
    idx_flat = idxmat[:n].T.reshape(-1)
    idx3 = jnp.pad(idx_flat, (0, 32 * trips * 128 - n_edges)).reshape(-1, 128)
    g = _gather_rows(table, idx3, n_edges)
    g3 = g.reshape(K, n, TW)

    pw1p = jnp.pad(params['pW1'], ((0, 13), (0, 0)))
    r1 = lambda v: v.reshape(1, 128)
    full = lambda a, b: pl.BlockSpec((a, b), lambda i: (0, 0))
    out = pl.pallas_call(
        _edge_body,
        grid=(n // BD,),
        in_specs=[
            pl.BlockSpec((BD, 128), lambda i: (i, 0)),
            pl.BlockSpec((BD, 16), lambda i: (i, 0)),
            pl.BlockSpec((K, BD, TW), lambda i: (0, i, 0)),
            full(128, 128),
            full(16, 128), full(1, 128), full(1, 128), full(1, 128),
            full(128, 128), full(1, 128), full(1, 128), full(1, 128),
            full(128, 128), full(1, 128), full(1, 128), full(1, 128),
            full(128, 128), full(1, 128), full(1, 128), full(1, 128),
        ],
        out_specs=pl.BlockSpec((BD, 128), lambda i: (i, 0)),
        out_shape=jax.ShapeDtypeStruct((n, 128), jnp.float32),
        scratch_shapes=[pltpu.VMEM((K, BD, 128), jnp.float32),
                        pltpu.VMEM((K, BD, 128), jnp.float32)],
    )(x, posp, g3, params['W_dst'],
      pw1p, r1(params['pb1']), r1(params['pg1']), r1(params['pbe1']),
      params['pW2'], r1(params['pb2']), r1(params['pg2']), r1(params['pbe2']),
      params['aW1'], r1(params['ab1']), r1(params['ag1']), r1(params['abe1']),
      params['aW2'], r1(params['ab2']), r1(params['ag2']), r1(params['abe2']))
    return out


# speedup vs baseline: 4.9952x; 4.9952x over previous
import functools

import jax
import jax.numpy as jnp
from jax import lax
from jax.experimental import pallas as pl
from jax.experimental.pallas import tpu as pltpu
from jax.experimental.pallas import tpu_sc as plsc

K = 16
NB = 8
BQ = 128
WT = 512
BD = 200
B1 = 200
TW = 384
BIG = 1e10


def _ln(t, g, b):
    m = jnp.mean(t, axis=-1, keepdims=True)
    d = t - m
    v = jnp.mean(d * d, axis=-1, keepdims=True)
    return d * lax.rsqrt(v + 1e-5) * g + b


def _elu(t):
    return jnp.where(t > 0, t, jnp.exp(jnp.minimum(t, 0.0)) - 1.0)


def _proj_body(x_ref, posp_ref, wl_ref, bl_ref, ws_ref, tab_ref):
    x = x_ref[...]
    xl = jnp.dot(x, wl_ref[...], preferred_element_type=jnp.float32) + bl_ref[...]
    asrc = jnp.dot(x, ws_ref[...], preferred_element_type=jnp.float32)
    tab_ref[:, 0:128] = xl
    tab_ref[:, 128:256] = asrc
    tab_ref[:, 256:272] = posp_ref[...]


def _knn_body(lo_ref, nt_ref, posq_ref, qb_ref, posT_ref, kb_ref,
              out_ref, d2_ref):
    i = pl.program_id(0)
    lo = lo_ref[i]
    nt = nt_ref[i]
    posq = posq_ref[...]
    sqq = jnp.sum(posq * posq, axis=1, keepdims=True)
    qb = qb_ref[...]
    qrow = lax.broadcasted_iota(jnp.int32, (BQ, 1), 0) + i * BQ

    def fill(t, _):
        pT = posT_ref[t]
        qk = jnp.dot(posq, pT, preferred_element_type=jnp.float32)
        sqk = jnp.sum(pT * pT, axis=0, keepdims=True)
        d2 = sqq + sqk - 2.0 * qk
        gcol = lax.broadcasted_iota(jnp.int32, (BQ, WT), 1) + t * WT
        bad = (kb_ref[t] != qb) | (gcol == qrow)
        d2_ref[t] = jnp.where(bad, jnp.float32(BIG), d2)
        return 0

    lax.fori_loop(lo, lo + nt, fill, 0)

    imax = jnp.int32(2**30)
    prevv = jnp.full((BQ, 1), -jnp.inf, jnp.float32)
    previ = jnp.full((BQ, 1), -1, jnp.int32)
    for k in range(K):
        def scan(t, carry, prevv=prevv, previ=previ):
            bv, bi = carry
            v = d2_ref[t]
            gcol = lax.broadcasted_iota(jnp.int32, (BQ, WT), 1) + t * WT
            elig = (v > prevv) | ((v == prevv) & (gcol > previ))
            vv = jnp.where(elig, v, jnp.inf)
            tmin = jnp.min(vv, axis=1, keepdims=True)
            ii = jnp.min(jnp.where(elig & (v == tmin), gcol, imax),
                         axis=1, keepdims=True)
            better = (tmin < bv) | ((tmin == bv) & (ii < bi))
            return (jnp.where(better, tmin, bv), jnp.where(better, ii, bi))

        bv, bi = lax.fori_loop(
            lo, lo + nt, scan,
            (jnp.full((BQ, 1), jnp.inf, jnp.float32),
             jnp.full((BQ, 1), imax, jnp.int32)))
        out_ref[:, k:k + 1] = bi
        prevv, previ = bv, bi


def _gather_rows(table, idx3, n_edges):
    rows = n_edges // 128
    trips = (rows + 31) // 32
    mesh = plsc.VectorSubcoreMesh(core_axis_name="c", subcore_axis_name="s")

    @functools.partial(
        pl.kernel, mesh=mesh,
        out_type=jax.ShapeDtypeStruct((n_edges, TW), jnp.float32),
        scratch_types=[
            pltpu.VMEM((trips, 128), jnp.int32),
            pltpu.VMEM((128, TW), jnp.float32),
            pltpu.SemaphoreType.DMA,
        ])
    def gk(tab_hbm, idx_hbm, out_hbm, idx_v, buf, sem):
        w = lax.axis_index("s") * 2 + lax.axis_index("c")
        pltpu.sync_copy(idx_hbm.at[pl.ds(w * trips, trips)], idx_v)
        nrow = jnp.minimum(trips, rows - w * trips)

        def body(j, _):
            row = w * trips + j
            pltpu.async_copy(tab_hbm.at[idx_v.at[j]], buf, sem).wait()
            pltpu.sync_copy(buf, out_hbm.at[pl.ds(row * 128, 128)])
            return 0

        lax.fori_loop(0, nrow, body, 0)

    return gk(table, idx3)


def _edge_body(x_ref, posp_ref, g_ref, wd_ref,
               pw1_ref, pb1_ref, pg1_ref, pbe1_ref,
               pw2_ref, pb2_ref, pg2_ref, pbe2_ref,
               aw1_ref, ab1_ref, ag1_ref, abe1_ref,
               aw2_ref, ab2_ref, ag2_ref, abe2_ref,
               out_ref, a_scr, m_scr):
    x = x_ref[...]
    adst = jnp.dot(x, wd_ref[...], preferred_element_type=jnp.float32)
    posp = posp_ref[...]

    def p1(k, rmax):
        gk = g_ref[k]
        gxl = gk[:, 0:128]
        gas = gk[:, 128:256]
        gpp = gk[:, 256:272]
        rel = posp - gpp
        h = _elu(_ln(jnp.dot(rel, pw1_ref[...],
                             preferred_element_type=jnp.float32) + pb1_ref[...],
                     pg1_ref[...], pbe1_ref[...]))
        delta = _elu(_ln(jnp.dot(h, pw2_ref[...],
                                 preferred_element_type=jnp.float32) + pb2_ref[...],
                         pg2_ref[...], pbe2_ref[...]))
        ai = adst - gas + delta
        h2 = _elu(_ln(jnp.dot(ai, aw1_ref[...],
                              preferred_element_type=jnp.float32) + ab1_ref[...],
                      ag1_ref[...], abe1_ref[...]))
        al = _elu(_ln(jnp.dot(h2, aw2_ref[...],
                              preferred_element_type=jnp.float32) + ab2_ref[...],
                      ag2_ref[...], abe2_ref[...]))
        a_scr[k] = al
        m_scr[k] = gxl + delta
        return jnp.maximum(rmax, al)

    rmax = lax.fori_loop(0, K, p1, jnp.full((BD, 128), -jnp.inf, jnp.float32))

    def p2(k, carry):
        s, acc = carry
        e = jnp.exp(a_scr[k] - rmax)
        return s + e, acc + e * m_scr[k]

    s, acc = lax.fori_loop(
        0, K, p2,
        (jnp.zeros((BD, 128), jnp.float32), jnp.zeros((BD, 128), jnp.float32)))
    out_ref[...] = acc / s


def kernel(x, pos, batch, params):
    n = x.shape[0]
    npad = ((n + WT - 1) // WT) * WT
    nq = npad // BQ
    nt = npad // WT

    batch_i = batch.astype(jnp.int32)
    posp = jnp.pad(pos, ((0, 0), (0, 13)))
    pos8 = jnp.pad(pos, ((0, npad - n), (0, 5)))
    posT3 = pos8.T.reshape(8, nt, WT).transpose(1, 0, 2)
    kb3 = jnp.pad(batch_i, (0, npad - n),
                  constant_values=-1).reshape(nt, 1, WT)
    qb2 = jnp.pad(batch_i, (0, npad - n),
                  constant_values=-2).reshape(npad, 1)

    seg = jnp.searchsorted(batch_i, jnp.arange(NB + 1, dtype=jnp.int32),
                           side='left').astype(jnp.int32)
    starts = jnp.arange(nq, dtype=jnp.int32) * BQ
    first = jnp.minimum(starts, n - 1)
    last = jnp.minimum(starts + BQ - 1, n - 1)
    clo = seg[batch_i[first]]
    chi = seg[batch_i[last] + 1]
    lo_t = clo // WT
    n_t = jnp.where(starts >= n, 0, (chi + WT - 1) // WT - lo_t)

    table = pl.pallas_call(
        _proj_body,
        grid=(n // B1,),
        in_specs=[
            pl.BlockSpec((B1, 128), lambda i: (i, 0)),
            pl.BlockSpec((B1, 16), lambda i: (i, 0)),
            pl.BlockSpec((128, 128), lambda i: (0, 0)),
            pl.BlockSpec((1, 128), lambda i: (0, 0)),
            pl.BlockSpec((128, 128), lambda i: (0, 0)),
        ],
        out_specs=pl.BlockSpec((B1, TW), lambda i: (i, 0)),
        out_shape=jax.ShapeDtypeStruct((n, TW), jnp.float32),
    )(x, posp, params['W_lin'], params['b_lin'].reshape(1, 128),
      params['W_src'])

    idxmat = pl.pallas_call(
        _knn_body,
        grid=(nq,),
        in_specs=[
            pl.BlockSpec(memory_space=pltpu.SMEM),
            pl.BlockSpec(memory_space=pltpu.SMEM),
            pl.BlockSpec((BQ, 8), lambda i: (i, 0)),
            pl.BlockSpec((BQ, 1), lambda i: (i, 0)),
            pl.BlockSpec((nt, 8, WT), lambda i: (0, 0, 0)),
            pl.BlockSpec((nt, 1, WT), lambda i: (0, 0, 0)),
        ],
        out_specs=pl.BlockSpec((BQ, K), lambda i: (i, 0)),
        out_shape=jax.ShapeDtypeStruct((npad, K), jnp.int32),
        scratch_shapes=[pltpu.VMEM((nt, BQ, WT), jnp.float32)],
    )(lo_t, n_t, pos8, qb2, posT3, kb3)

    n_edges = K * n
    rows = n_edges // 128
    trips = (rows + 31) // 32
    idx_flat = idxmat[:n].T.reshape(-1)
    idx3 = jnp.pad(idx_flat, (0, 32 * trips * 128 - n_edges)).reshape(-1, 128)
    g = _gather_rows(table, idx3, n_edges)
    g3 = g.reshape(K, n, TW)

    pw1p = jnp.pad(params['pW1'], ((0, 13), (0, 0)))
    r1 = lambda v: v.reshape(1, 128)
    full = lambda a, b: pl.BlockSpec((a, b), lambda i: (0, 0))
    out = pl.pallas_call(
        _edge_body,
        grid=(n // BD,),
        in_specs=[
            pl.BlockSpec((BD, 128), lambda i: (i, 0)),
            pl.BlockSpec((BD, 16), lambda i: (i, 0)),
            pl.BlockSpec((K, BD, TW), lambda i: (0, i, 0)),
            full(128, 128),
            full(16, 128), full(1, 128), full(1, 128), full(1, 128),
            full(128, 128), full(1, 128), full(1, 128), full(1, 128),
            full(128, 128), full(1, 128), full(1, 128), full(1, 128),
            full(128, 128), full(1, 128), full(1, 128), full(1, 128),
        ],
        out_specs=pl.BlockSpec((BD, 128), lambda i: (i, 0)),
        out_shape=jax.ShapeDtypeStruct((n, 128), jnp.float32),
        scratch_shapes=[pltpu.VMEM((K, BD, 128), jnp.float32),
                        pltpu.VMEM((K, BD, 128), jnp.float32)],
    )(x, posp, g3, params['W_dst'],
      pw1p, r1(params['pb1']), r1(params['pg1']), r1(params['pbe1']),
      params['pW2'], r1(params['pb2']), r1(params['pg2']), r1(params['pbe2']),
      params['aW1'], r1(params['ab1']), r1(params['ag1']), r1(params['abe1']),
      params['aW2'], r1(params['ab2']), r1(params['ag2']), r1(params['abe2']))
    return out
